# baseline (speedup 1.0000x reference)
import functools
import jax
import jax.numpy as jnp
from jax import lax
from jax.experimental import pallas as pl
from jax.experimental.pallas import tpu as pltpu
from jax.experimental.pallas import tpu_sc as plsc

N = 10000
E = 320000
D = 128

NC = 2
NS = 16
NW = NC * NS
EPW = E // NW
GC = 125
GG = 40
NG = EPW // (GC * GG)
SGC = 80
SGG = 25
SNG = EPW // (SGC * SGG)
NPAD = 10240
RPT = NPAD // NS
RLAST = N - 15 * RPT

_mesh = plsc.VectorSubcoreMesh(core_axis_name="c", subcore_axis_name="s")


@functools.partial(
    pl.kernel,
    out_type=jax.ShapeDtypeStruct((NC, N, D), jnp.float32),
    mesh=_mesh,
    scratch_types=[
        pltpu.VMEM((GG, 1, GC), jnp.int32),
        pltpu.VMEM((GC, D), jnp.float32),
        pltpu.VMEM_SHARED((N, D), jnp.float32),
        pltpu.SemaphoreType.DMA,
    ],
)
def _sc_deg(ones_rows, dst4, zeros, out, dst_v, rows_v, acc, sem):
    cid = lax.axis_index("c")
    sid = lax.axis_index("s")
    wid = sid * NC + cid

    @pl.when(sid < NS - 1)
    def _():
        pltpu.sync_copy(zeros.at[pl.ds(sid * RPT, RPT)],
                        acc.at[pl.ds(sid * RPT, RPT)])

    @pl.when(sid == NS - 1)
    def _():
        pltpu.sync_copy(zeros.at[pl.ds(15 * RPT, RLAST)],
                        acc.at[pl.ds(15 * RPT, RLAST)])

    plsc.subcore_barrier()

    def group(grp, carry):
        pltpu.sync_copy(dst4.at[wid, grp], dst_v)

        @pl.when(grp == 0)
        def _():
            pltpu.async_copy(ones_rows.at[dst_v.at[0, 0]], rows_v, sem).wait()

        for j in range(GG):
            pltpu.sync_copy(rows_v, acc.at[dst_v.at[j, 0]], add=True)
        return carry

    lax.fori_loop(0, NG, group, 0)
    plsc.subcore_barrier()

    @pl.when(sid < NS - 1)
    def _():
        pltpu.sync_copy(acc.at[pl.ds(sid * RPT, RPT)],
                        out.at[cid, pl.ds(sid * RPT, RPT)])

    @pl.when(sid == NS - 1)
    def _():
        pltpu.sync_copy(acc.at[pl.ds(15 * RPT, RLAST)],
                        out.at[cid, pl.ds(15 * RPT, RLAST)])


@functools.partial(
    pl.kernel,
    out_type=jax.ShapeDtypeStruct((NC, N, D), jnp.float32),
    mesh=_mesh,
    scratch_types=[
        pltpu.VMEM((SGG, 1, SGC), jnp.int32),
        pltpu.VMEM((SGG, 1, SGC), jnp.int32),
        pltpu.VMEM((SGC, D), jnp.float32),
        pltpu.VMEM((SGC, D), jnp.float32),
        pltpu.VMEM((SGC, D), jnp.float32),
        pltpu.VMEM((SGC, D), jnp.float32),
        pltpu.VMEM_SHARED((N, D), jnp.float32),
        pltpu.SemaphoreType.DMA,
        pltpu.SemaphoreType.DMA,
        pltpu.SemaphoreType.DMA,
        pltpu.SemaphoreType.DMA,
    ],
)
def _sc_scatter(g, src4, dst4, zeros, out, src_v, dst_v, row0, row1, row2,
                row3, acc, sem0, sem1, sem2, sem3):
    cid = lax.axis_index("c")
    sid = lax.axis_index("s")
    wid = sid * NC + cid

    @pl.when(sid < NS - 1)
    def _():
        pltpu.sync_copy(zeros.at[pl.ds(sid * RPT, RPT)],
                        acc.at[pl.ds(sid * RPT, RPT)])

    @pl.when(sid == NS - 1)
    def _():
        pltpu.sync_copy(zeros.at[pl.ds(15 * RPT, RLAST)],
                        acc.at[pl.ds(15 * RPT, RLAST)])

    plsc.subcore_barrier()

    bufs = (row0, row1, row2, row3)
    sems = (sem0, sem1, sem2, sem3)

    def group(grp, carry):
        pltpu.sync_copy(src4.at[wid, grp], src_v)
        pltpu.sync_copy(dst4.at[wid, grp], dst_v)
        for j in range(3):
            pltpu.async_copy(g.at[src_v.at[j, 0]], bufs[j], sems[j])
        for j in range(SGG):
            b = j % 4
            if j + 3 < SGG:
                nb = (j + 3) % 4
                pltpu.async_copy(g.at[src_v.at[j + 3, 0]], bufs[nb],
                                 sems[nb])
            pltpu.make_async_copy(g.at[src_v.at[j, 0]], bufs[b],
                                  sems[b]).wait()
            pltpu.sync_copy(bufs[b], acc.at[dst_v.at[j, 0]], add=True)
        return carry

    lax.fori_loop(0, SNG, group, 0)
    plsc.subcore_barrier()

    @pl.when(sid < NS - 1)
    def _():
        pltpu.sync_copy(acc.at[pl.ds(sid * RPT, RPT)],
                        out.at[cid, pl.ds(sid * RPT, RPT)])

    @pl.when(sid == NS - 1)
    def _():
        pltpu.sync_copy(acc.at[pl.ds(15 * RPT, RLAST)],
                        out.at[cid, pl.ds(15 * RPT, RLAST)])


_R = 2000


def _dinv_block(deg2_ref):
    deg = deg2_ref[0, :, 0:1] + deg2_ref[1, :, 0:1] + 1.0
    return lax.rsqrt(deg)


def _tc_prep_body(x_ref, w_ref, deg2_ref, g_ref, dinv8_ref):
    dinv = _dinv_block(deg2_ref)
    hw = jnp.dot(x_ref[...], w_ref[...],
                 preferred_element_type=jnp.float32,
                 precision=lax.Precision.HIGHEST)
    g_ref[...] = dinv * hw
    dinv8_ref[...] = jnp.broadcast_to(dinv, (_R, 8))


def _tc_mid_body(p_ref, gp_ref, dinv8_ref, b_ref, w_ref, g_ref):
    dinv = dinv8_ref[:, 0:1]
    agg = (p_ref[0] + p_ref[1] + gp_ref[...]) * dinv + b_ref[...]
    h = jnp.maximum(agg, 0.0)
    hw = jnp.dot(h, w_ref[...],
                 preferred_element_type=jnp.float32,
                 precision=lax.Precision.HIGHEST)
    g_ref[...] = dinv * hw


def _tc_final_body(p_ref, gp_ref, dinv8_ref, b_ref, out_ref):
    dinv = dinv8_ref[:, 0:1]
    out_ref[...] = (p_ref[0] + p_ref[1] + gp_ref[...]) * dinv + b_ref[...]


_row_spec = pl.BlockSpec((_R, D), lambda i: (i, 0))
_p_spec = pl.BlockSpec((NC, _R, D), lambda i: (0, i, 0))
_w_spec = pl.BlockSpec((D, D), lambda i: (0, 0))
_b_spec = pl.BlockSpec((1, D), lambda i: (0, 0))
_dinv_spec = pl.BlockSpec((_R, 8), lambda i: (i, 0))
_out_shape = jax.ShapeDtypeStruct((N, D), jnp.float32)

_tc_prep = pl.pallas_call(
    _tc_prep_body,
    grid=(N // _R,),
    in_specs=[_row_spec, _w_spec, _p_spec],
    out_specs=[_row_spec, _dinv_spec],
    out_shape=[_out_shape, jax.ShapeDtypeStruct((N, 8), jnp.float32)],
)

_tc_mid = pl.pallas_call(
    _tc_mid_body,
    grid=(N // _R,),
    in_specs=[_p_spec, _row_spec, _dinv_spec, _b_spec, _w_spec],
    out_specs=_row_spec,
    out_shape=_out_shape,
)

_tc_final = pl.pallas_call(
    _tc_final_body,
    grid=(N // _R,),
    in_specs=[_p_spec, _row_spec, _dinv_spec, _b_spec],
    out_specs=_row_spec,
    out_shape=_out_shape,
)


def kernel(x, edge_index, W1, b1, W2, b2, W3, b3):
    src4 = edge_index[0].reshape(NW, SNG, SGG, 1, SGC)
    dst4 = edge_index[1].reshape(NW, SNG, SGG, 1, SGC)
    dst4d = edge_index[1].reshape(NW, NG, GG, 1, GC)
    zeros = jnp.zeros((N, D), jnp.float32)
    ones_rows = jnp.ones((N, D), jnp.float32)
    b1r = b1.reshape(1, D)
    b2r = b2.reshape(1, D)
    b3r = b3.reshape(1, D)

    deg2 = _sc_deg(ones_rows, dst4d, zeros)
    g1, dinv8 = _tc_prep(x, W1, deg2)
    p1 = _sc_scatter(g1, src4, dst4, zeros)
    g2 = _tc_mid(p1, g1, dinv8, b1r, W2)
    p2 = _sc_scatter(g2, src4, dst4, zeros)
    g3 = _tc_mid(p2, g2, dinv8, b2r, W3)
    p3 = _sc_scatter(g3, src4, dst4, zeros)
    out = _tc_final(p3, g3, dinv8, b3r)
    return out


# speedup vs baseline: 1.0169x; 1.0169x over previous
import functools
import jax
import jax.numpy as jnp
from jax import lax
from jax.experimental import pallas as pl
from jax.experimental.pallas import tpu as pltpu
from jax.experimental.pallas import tpu_sc as plsc

N = 10000
E = 320000
D = 128

NC = 2
NS = 16
NW = NC * NS
EPW = E // NW
GC = 125
GG = 40
NG = EPW // (GC * GG)
SGC = 80
SGG = 25
SNG = EPW // (SGC * SGG)
NPAD = 10240
RPT = NPAD // NS
RLAST = N - 15 * RPT

_mesh = plsc.VectorSubcoreMesh(core_axis_name="c", subcore_axis_name="s")


@functools.partial(
    pl.kernel,
    out_type=jax.ShapeDtypeStruct((NC, N, D), jnp.float32),
    mesh=_mesh,
    scratch_types=[
        pltpu.VMEM((GG, 1, GC), jnp.int32),
        pltpu.VMEM((GC, D), jnp.float32),
        pltpu.VMEM_SHARED((N, D), jnp.float32),
        pltpu.SemaphoreType.DMA,
    ],
)
def _sc_deg(ones_rows, dst4, zeros, out, dst_v, rows_v, acc, sem):
    cid = lax.axis_index("c")
    sid = lax.axis_index("s")
    wid = sid * NC + cid

    @pl.when(sid < NS - 1)
    def _():
        pltpu.sync_copy(zeros.at[pl.ds(sid * RPT, RPT)],
                        acc.at[pl.ds(sid * RPT, RPT)])

    @pl.when(sid == NS - 1)
    def _():
        pltpu.sync_copy(zeros.at[pl.ds(15 * RPT, RLAST)],
                        acc.at[pl.ds(15 * RPT, RLAST)])

    plsc.subcore_barrier()

    def group(grp, carry):
        pltpu.sync_copy(dst4.at[wid, grp], dst_v)

        @pl.when(grp == 0)
        def _():
            pltpu.async_copy(ones_rows.at[dst_v.at[0, 0]], rows_v, sem).wait()

        for j in range(GG):
            pltpu.sync_copy(rows_v, acc.at[dst_v.at[j, 0]], add=True)
        return carry

    lax.fori_loop(0, NG, group, 0)
    plsc.subcore_barrier()

    @pl.when(sid < NS - 1)
    def _():
        pltpu.sync_copy(acc.at[pl.ds(sid * RPT, RPT)],
                        out.at[cid, pl.ds(sid * RPT, RPT)])

    @pl.when(sid == NS - 1)
    def _():
        pltpu.sync_copy(acc.at[pl.ds(15 * RPT, RLAST)],
                        out.at[cid, pl.ds(15 * RPT, RLAST)])


@functools.partial(
    pl.kernel,
    out_type=jax.ShapeDtypeStruct((NC, N, D), jnp.float32),
    mesh=_mesh,
    scratch_types=[
        pltpu.VMEM((SGG, 1, SGC), jnp.int32),
        pltpu.VMEM((SGG, 1, SGC), jnp.int32),
        pltpu.VMEM((SGC, D), jnp.float32),
        pltpu.VMEM((SGC, D), jnp.float32),
        pltpu.VMEM((SGC, D), jnp.float32),
        pltpu.VMEM((SGC, D), jnp.float32),
        pltpu.VMEM_SHARED((N, D), jnp.float32),
        pltpu.SemaphoreType.DMA,
        pltpu.SemaphoreType.DMA,
        pltpu.SemaphoreType.DMA,
        pltpu.SemaphoreType.DMA,
    ],
)
def _sc_scatter(g, src4, dst4, zeros, out, src_v, dst_v, row0, row1, row2,
                row3, acc, sem0, sem1, sem2, sem3):
    cid = lax.axis_index("c")
    sid = lax.axis_index("s")
    wid = sid * NC + cid

    @pl.when(sid < NS - 1)
    def _():
        pltpu.sync_copy(zeros.at[pl.ds(sid * RPT, RPT)],
                        acc.at[pl.ds(sid * RPT, RPT)])

    @pl.when(sid == NS - 1)
    def _():
        pltpu.sync_copy(zeros.at[pl.ds(15 * RPT, RLAST)],
                        acc.at[pl.ds(15 * RPT, RLAST)])

    plsc.subcore_barrier()

    bufs = (row0, row1, row2, row3)
    sems = (sem0, sem1, sem2, sem3)

    def group(grp, carry):
        pltpu.sync_copy(src4.at[wid, grp], src_v)
        pltpu.sync_copy(dst4.at[wid, grp], dst_v)
        for j in range(3):
            pltpu.async_copy(g.at[src_v.at[j, 0]], bufs[j], sems[j])
        for j in range(SGG):
            b = j % 4
            pltpu.make_async_copy(g.at[src_v.at[j, 0]], bufs[b],
                                  sems[b]).wait()
            if j + 3 < SGG:
                nb = (j + 3) % 4
                pltpu.async_copy(g.at[src_v.at[j + 3, 0]], bufs[nb],
                                 sems[nb])
            pltpu.sync_copy(bufs[b], acc.at[dst_v.at[j, 0]], add=True)
        return carry

    lax.fori_loop(0, SNG, group, 0)
    plsc.subcore_barrier()

    @pl.when(sid < NS - 1)
    def _():
        pltpu.sync_copy(acc.at[pl.ds(sid * RPT, RPT)],
                        out.at[cid, pl.ds(sid * RPT, RPT)])

    @pl.when(sid == NS - 1)
    def _():
        pltpu.sync_copy(acc.at[pl.ds(15 * RPT, RLAST)],
                        out.at[cid, pl.ds(15 * RPT, RLAST)])


_R = 2000


def _dinv_block(deg2_ref):
    deg = deg2_ref[0, :, 0:1] + deg2_ref[1, :, 0:1] + 1.0
    return lax.rsqrt(deg)


def _tc_prep_body(x_ref, w_ref, deg2_ref, g_ref, dinv8_ref):
    dinv = _dinv_block(deg2_ref)
    hw = jnp.dot(x_ref[...], w_ref[...],
                 preferred_element_type=jnp.float32,
                 precision=lax.Precision.HIGHEST)
    g_ref[...] = dinv * hw
    dinv8_ref[...] = jnp.broadcast_to(dinv, (_R, 8))


def _tc_mid_body(p_ref, gp_ref, dinv8_ref, b_ref, w_ref, g_ref):
    dinv = dinv8_ref[:, 0:1]
    agg = (p_ref[0] + p_ref[1] + gp_ref[...]) * dinv + b_ref[...]
    h = jnp.maximum(agg, 0.0)
    hw = jnp.dot(h, w_ref[...],
                 preferred_element_type=jnp.float32,
                 precision=lax.Precision.HIGHEST)
    g_ref[...] = dinv * hw


def _tc_final_body(p_ref, gp_ref, dinv8_ref, b_ref, out_ref):
    dinv = dinv8_ref[:, 0:1]
    out_ref[...] = (p_ref[0] + p_ref[1] + gp_ref[...]) * dinv + b_ref[...]


_row_spec = pl.BlockSpec((_R, D), lambda i: (i, 0))
_p_spec = pl.BlockSpec((NC, _R, D), lambda i: (0, i, 0))
_w_spec = pl.BlockSpec((D, D), lambda i: (0, 0))
_b_spec = pl.BlockSpec((1, D), lambda i: (0, 0))
_dinv_spec = pl.BlockSpec((_R, 8), lambda i: (i, 0))
_out_shape = jax.ShapeDtypeStruct((N, D), jnp.float32)

_tc_prep = pl.pallas_call(
    _tc_prep_body,
    grid=(N // _R,),
    in_specs=[_row_spec, _w_spec, _p_spec],
    out_specs=[_row_spec, _dinv_spec],
    out_shape=[_out_shape, jax.ShapeDtypeStruct((N, 8), jnp.float32)],
)

_tc_mid = pl.pallas_call(
    _tc_mid_body,
    grid=(N // _R,),
    in_specs=[_p_spec, _row_spec, _dinv_spec, _b_spec, _w_spec],
    out_specs=_row_spec,
    out_shape=_out_shape,
)

_tc_final = pl.pallas_call(
    _tc_final_body,
    grid=(N // _R,),
    in_specs=[_p_spec, _row_spec, _dinv_spec, _b_spec],
    out_specs=_row_spec,
    out_shape=_out_shape,
)


def kernel(x, edge_index, W1, b1, W2, b2, W3, b3):
    src4 = edge_index[0].reshape(NW, SNG, SGG, 1, SGC)
    dst4 = edge_index[1].reshape(NW, SNG, SGG, 1, SGC)
    dst4d = edge_index[1].reshape(NW, NG, GG, 1, GC)
    zeros = jnp.zeros((N, D), jnp.float32)
    ones_rows = jnp.ones((N, D), jnp.float32)
    b1r = b1.reshape(1, D)
    b2r = b2.reshape(1, D)
    b3r = b3.reshape(1, D)

    deg2 = _sc_deg(ones_rows, dst4d, zeros)
    g1, dinv8 = _tc_prep(x, W1, deg2)
    p1 = _sc_scatter(g1, src4, dst4, zeros)
    g2 = _tc_mid(p1, g1, dinv8, b1r, W2)
    p2 = _sc_scatter(g2, src4, dst4, zeros)
    g3 = _tc_mid(p2, g2, dinv8, b2r, W3)
    p3 = _sc_scatter(g3, src4, dst4, zeros)
    out = _tc_final(p3, g3, dinv8, b3r)
    return out
